# Initial kernel scaffold; baseline (speedup 1.0000x reference)
import jax
import jax.numpy as jnp
from jax.experimental import pallas as pl


def kernel(hidden_states, Wg, bg, We, be):
    raise NotImplementedError("write your pallas kernel here")


# speedup vs baseline: 1.2249x; 1.2249x over previous
import functools

import jax
import jax.numpy as jnp
from jax import lax
from jax.experimental import pallas as pl
from jax.experimental.pallas import tpu as pltpu
from jax.experimental.pallas import tpu_sc as plsc

NUM_EXPERTS = 64
D_MODEL = 8
N_TOKENS = 32768

NC = 2
NS = 16
L = 16
NW = NC * NS
TPW = N_TOKENS // NW
GROUPS = TPW // L
PAD = 128


def _gate_body(x_ref, wg_ref, bg_ref, e1_ref, e2_ref):
    logits = (
        jnp.dot(x_ref[...], wg_ref[...], preferred_element_type=jnp.float32)
        + bg_ref[...]
    )
    iota = lax.broadcasted_iota(jnp.int32, logits.shape, 1)
    m1 = jnp.max(logits, axis=1, keepdims=True)
    e1 = jnp.min(jnp.where(logits == m1, iota, NUM_EXPERTS), axis=1)
    masked = jnp.where(iota == e1[:, None], -jnp.inf, logits)
    m2 = jnp.max(masked, axis=1, keepdims=True)
    e2 = jnp.min(jnp.where(masked == m2, iota, NUM_EXPERTS), axis=1)
    e1_ref[...] = e1
    e2_ref[...] = e2


def _gate_topk(x, Wg, bg, *, interpret=False):
    blk = 2048
    grid = N_TOKENS // blk
    return pl.pallas_call(
        _gate_body,
        grid=(grid,),
        in_specs=[
            pl.BlockSpec((blk, D_MODEL), lambda i: (i, 0)),
            pl.BlockSpec((D_MODEL, NUM_EXPERTS), lambda i: (0, 0)),
            pl.BlockSpec((1, NUM_EXPERTS), lambda i: (0, 0)),
        ],
        out_specs=[
            pl.BlockSpec((blk,), lambda i: (i,)),
            pl.BlockSpec((blk,), lambda i: (i,)),
        ],
        out_shape=[
            jax.ShapeDtypeStruct((N_TOKENS,), jnp.int32),
            jax.ShapeDtypeStruct((N_TOKENS,), jnp.int32),
        ],
        interpret=interpret,
    )(x, Wg, bg.reshape(1, NUM_EXPERTS))


def _sc_dispatch_body(
    x_hbm, e1_hbm, e2_hbm, we_hbm, be_hbm, out_hbm,
    x_v, out_v, we_v, be_v, e1_v, e2_v,
):
    wid = lax.axis_index("s") * NC + lax.axis_index("c")
    tok0 = wid * TPW
    pltpu.sync_copy(x_hbm.at[pl.ds(tok0 * D_MODEL, TPW * D_MODEL)],
                    x_v.at[pl.ds(0, TPW * D_MODEL)])
    pltpu.sync_copy(e1_hbm.at[pl.ds(tok0, TPW)], e1_v)
    pltpu.sync_copy(e2_hbm.at[pl.ds(tok0, TPW)], e2_v)
    pltpu.sync_copy(we_hbm, we_v)
    pltpu.sync_copy(be_hbm, be_v)

    iota8 = lax.iota(jnp.int32, L) * D_MODEL
    iota_d = [iota8 + d for d in range(D_MODEL)]

    def body(g, carry):
        t0 = g * L
        o0 = t0 * D_MODEL
        i1 = e1_v[pl.ds(t0, L)]
        i2 = e2_v[pl.ds(t0, L)]
        b1 = i1 * (D_MODEL * D_MODEL)
        b2 = i2 * (D_MODEL * D_MODEL)
        c1 = i1 * D_MODEL
        c2 = i2 * D_MODEL
        accs = []
        for j in range(D_MODEL):
            a1 = plsc.load_gather(be_v, [c1 + j])
            a2 = plsc.load_gather(be_v, [c2 + j])
            accs.append(a1 + a2)
        for k in range(D_MODEL):
            xk = plsc.load_gather(x_v.at[pl.ds(o0, PAD)], [iota_d[k]])
            for j in range(D_MODEL):
                off = k * D_MODEL + j
                w1 = plsc.load_gather(we_v, [b1 + off])
                w2 = plsc.load_gather(we_v, [b2 + off])
                accs[j] = accs[j] + xk * (w1 + w2)
        for j in range(D_MODEL):
            plsc.store_scatter(out_v.at[pl.ds(o0, PAD)], [iota_d[j]], accs[j])
        return carry

    lax.fori_loop(0, GROUPS, body, 0)
    pltpu.sync_copy(out_v.at[pl.ds(0, TPW * D_MODEL)],
                    out_hbm.at[pl.ds(tok0 * D_MODEL, TPW * D_MODEL)])


@functools.lru_cache(maxsize=1)
def _sc_dispatch():
    return pl.kernel(
        _sc_dispatch_body,
        out_type=jax.ShapeDtypeStruct((N_TOKENS * D_MODEL,), jnp.float32),
        mesh=plsc.VectorSubcoreMesh(
            core_axis_name="c", subcore_axis_name="s", num_cores=NC, num_subcores=NS
        ),
        compiler_params=pltpu.CompilerParams(needs_layout_passes=False),
        scratch_types=[
            pltpu.VMEM((TPW * D_MODEL + PAD,), jnp.float32),
            pltpu.VMEM((TPW * D_MODEL + PAD,), jnp.float32),
            pltpu.VMEM((NUM_EXPERTS * D_MODEL * D_MODEL,), jnp.float32),
            pltpu.VMEM((NUM_EXPERTS * D_MODEL,), jnp.float32),
            pltpu.VMEM((TPW,), jnp.int32),
            pltpu.VMEM((TPW,), jnp.int32),
        ],
    )


@jax.jit
def kernel(hidden_states, Wg, bg, We, be):
    e1, e2 = _gate_topk(hidden_states, Wg, bg)
    out_flat = _sc_dispatch()(
        hidden_states.reshape(-1), e1, e2, We.reshape(-1), be.reshape(-1)
    )
    return out_flat.reshape(N_TOKENS, D_MODEL)


# speedup vs baseline: 3.9900x; 3.2574x over previous
import functools

import jax
import jax.numpy as jnp
from jax import lax
from jax.experimental import pallas as pl
from jax.experimental.pallas import tpu as pltpu
from jax.experimental.pallas import tpu_sc as plsc

E = 64
D = 8
N = 32768

NC = 2
NS = 16
L = 16
NW = NC * NS
TPW = N // NW
GROUPS = TPW // L

WE_PITCH = E * D * D + 1
BE_PITCH = E * D + 1


def _gate_body(xt_ref, wgt_ref, bg_ref, e1_ref, e2_ref):
    logits = (
        jnp.dot(wgt_ref[...], xt_ref[...], preferred_element_type=jnp.float32)
        + bg_ref[...]
    )
    iota = lax.broadcasted_iota(jnp.int32, logits.shape, 0)
    m1 = jnp.max(logits, axis=0, keepdims=True)
    e1 = jnp.min(jnp.where(logits == m1, iota, E), axis=0)
    masked = jnp.where(iota == e1[None, :], -jnp.inf, logits)
    m2 = jnp.max(masked, axis=0, keepdims=True)
    e2 = jnp.min(jnp.where(masked == m2, iota, E), axis=0)
    e1_ref[...] = e1
    e2_ref[...] = e2


def _gate_topk(xt, WgT, bg2d, *, interpret=False):
    blk = 4096
    grid = N // blk
    return pl.pallas_call(
        _gate_body,
        grid=(grid,),
        in_specs=[
            pl.BlockSpec((D, blk), lambda i: (0, i)),
            pl.BlockSpec((E, D), lambda i: (0, 0)),
            pl.BlockSpec((E, 1), lambda i: (0, 0)),
        ],
        out_specs=[
            pl.BlockSpec((blk,), lambda i: (i,)),
            pl.BlockSpec((blk,), lambda i: (i,)),
        ],
        out_shape=[
            jax.ShapeDtypeStruct((N,), jnp.int32),
            jax.ShapeDtypeStruct((N,), jnp.int32),
        ],
        interpret=interpret,
    )(xt, WgT, bg2d)


def _sc_dispatch_body(
    xt_hbm, e1_hbm, e2_hbm, we_hbm, be_hbm, out_hbm,
    xt_v, out_v, we_v, be_v, e1_v, e2_v, sem,
):
    wid = lax.axis_index("s") * NC + lax.axis_index("c")
    tok0 = wid * TPW

    copies = []
    for k in range(D):
        copies.append(pltpu.async_copy(
            xt_hbm.at[pl.ds(k * N + tok0, TPW)],
            xt_v.at[pl.ds(k * TPW, TPW)], sem))
    copies.append(pltpu.async_copy(e1_hbm.at[pl.ds(tok0, TPW)], e1_v, sem))
    copies.append(pltpu.async_copy(e2_hbm.at[pl.ds(tok0, TPW)], e2_v, sem))
    copies.append(pltpu.async_copy(we_hbm, we_v, sem))
    copies.append(pltpu.async_copy(be_hbm, be_v, sem))
    for c in copies:
        c.wait()

    iota = lax.iota(jnp.int32, L)
    base_we = iota * WE_PITCH
    base_be = iota * BE_PITCH

    def body(g, carry):
        t0 = g * L
        i1 = e1_v[pl.ds(t0, L)]
        i2 = e2_v[pl.ds(t0, L)]
        q1 = base_we + i1 * (D * D)
        q2 = base_we + i2 * (D * D)
        c1 = base_be + i1 * D
        c2 = base_be + i2 * D
        accs = []
        for j in range(D):
            a1 = plsc.load_gather(be_v, [c1 + j])
            a2 = plsc.load_gather(be_v, [c2 + j])
            accs.append(a1 + a2)
        for k in range(D):
            xk = xt_v[pl.ds(k * TPW + t0, L)]
            for j in range(D):
                off = k * D + j
                w1 = plsc.load_gather(we_v, [q1 + off])
                w2 = plsc.load_gather(we_v, [q2 + off])
                accs[j] = accs[j] + xk * (w1 + w2)
        for j in range(D):
            out_v[pl.ds(j * TPW + t0, L)] = accs[j]
        return carry

    lax.fori_loop(0, GROUPS, body, 0)

    outs = []
    for j in range(D):
        outs.append(pltpu.async_copy(
            out_v.at[pl.ds(j * TPW, TPW)],
            out_hbm.at[pl.ds(j * N + tok0, TPW)], sem))
    for c in outs:
        c.wait()


@functools.lru_cache(maxsize=1)
def _sc_dispatch():
    return pl.kernel(
        _sc_dispatch_body,
        out_type=jax.ShapeDtypeStruct((D * N,), jnp.float32),
        mesh=plsc.VectorSubcoreMesh(
            core_axis_name="c", subcore_axis_name="s", num_cores=NC, num_subcores=NS
        ),
        compiler_params=pltpu.CompilerParams(needs_layout_passes=False),
        scratch_types=[
            pltpu.VMEM((TPW * D,), jnp.float32),
            pltpu.VMEM((TPW * D,), jnp.float32),
            pltpu.VMEM((L * WE_PITCH,), jnp.float32),
            pltpu.VMEM((L * BE_PITCH,), jnp.float32),
            pltpu.VMEM((TPW,), jnp.int32),
            pltpu.VMEM((TPW,), jnp.int32),
            pltpu.SemaphoreType.DMA,
        ],
    )


@jax.jit
def kernel(hidden_states, Wg, bg, We, be):
    xt = hidden_states.T.reshape(-1)
    e1, e2 = _gate_topk(
        hidden_states.T, Wg.T, bg.reshape(E, 1)
    )
    we_rep = jnp.pad(
        jnp.broadcast_to(We.reshape(1, E * D * D), (L, E * D * D)),
        ((0, 0), (0, 1)),
    ).reshape(-1)
    be_rep = jnp.pad(
        jnp.broadcast_to(be.reshape(1, E * D), (L, E * D)), ((0, 0), (0, 1))
    ).reshape(-1)
    out_t = _sc_dispatch()(xt, e1, e2, we_rep, be_rep)
    return out_t.reshape(D, N).T
